# initial kernel scaffold (unmeasured)
import jax
import jax.numpy as jnp
from jax import lax
from jax.experimental import pallas as pl
from jax.experimental.pallas import tpu as pltpu

N_DEV = 4
N_EXP = 16
EXP_PER_DEV = 4
CAPACITY = 204


def kernel(x, router_W, route_idx, expert_W):
    del router_W
    n_tok, d_model = x.shape
    _, _, d_ff = expert_W.shape

    xb = x.astype(jnp.bfloat16)
    wb = expert_W.astype(jnp.bfloat16)
    route_f = route_idx.astype(jnp.float32)
    oh = (route_idx == jnp.arange(N_EXP, dtype=jnp.int32)[None, :]).astype(
        jnp.float32
    )
    rank = jnp.take_along_axis(
        jnp.cumsum(oh, axis=0) - oh, route_idx, axis=1
    )
    cnts = jnp.zeros((1, 128), jnp.float32).at[0, :N_EXP].set(oh.sum(axis=0))

    def body(
        x_ref, w_ref, route_ref, oh_ref, rank_ref, cnt_ref, out_ref,
        comm_ref, send_sems, recv_sems, cnt_send_sems, cnt_recv_sems,
        counts_all,
    ):
        my = lax.axis_index("i")
        right = jnp.mod(my + 1, N_DEV)

        barrier = pltpu.get_barrier_semaphore()
        for o in range(1, N_DEV):
            pl.semaphore_signal(
                barrier, inc=1,
                device_id=(jnp.mod(my + o, N_DEV),),
                device_id_type=pl.DeviceIdType.MESH,
            )
        pl.semaphore_wait(barrier, N_DEV - 1)

        cnt_sends = []
        for o in range(1, N_DEV):
            tgt = jnp.mod(my + o, N_DEV)
            s = pltpu.make_async_remote_copy(
                src_ref=cnt_ref,
                dst_ref=counts_all.at[pl.ds(my, 1)],
                send_sem=cnt_send_sems.at[o],
                recv_sem=cnt_recv_sems.at[my],
                device_id=(tgt,),
                device_id_type=pl.DeviceIdType.MESH,
            )
            s.start()
            cnt_sends.append(s)
        for o in range(1, N_DEV):
            src_dev = jnp.mod(my + o, N_DEV)
            r = pltpu.make_async_remote_copy(
                src_ref=cnt_ref,
                dst_ref=counts_all.at[pl.ds(src_dev, 1)],
                send_sem=cnt_send_sems.at[0],
                recv_sem=cnt_recv_sems.at[src_dev],
                device_id=(src_dev,),
                device_id_type=pl.DeviceIdType.MESH,
            )
            r.wait_recv()
        for s in cnt_sends:
            s.wait_send()

        ca = counts_all[...]
        dev_iota = lax.broadcasted_iota(jnp.int32, (N_DEV, 128), 0)
        prior = jnp.where(dev_iota < my, ca, 0.0).sum(
            axis=0, keepdims=True
        )
        offs_tok = (oh_ref[...] * prior[:, :N_EXP]).sum(
            axis=1, keepdims=True
        )
        keep = (offs_tok + rank_ref[...]) < float(CAPACITY)

        xv = x_ref[...]
        routev = route_ref[...]
        acc = jnp.zeros((n_tok, d_ff), jnp.float32)

        def contrib(chunk, src_dev, acc):
            for k in range(EXP_PER_DEV):
                e = (src_dev * EXP_PER_DEV + k).astype(jnp.float32)
                m = jnp.logical_and(routev == e, keep)
                xm = xv * m.astype(jnp.bfloat16)
                acc = acc + jnp.dot(
                    xm, chunk[k], preferred_element_type=jnp.float32
                )
            return acc

        for h in range(N_DEV):
            chunk_ref = w_ref if h == 0 else comm_ref.at[h - 1]
            if h < N_DEV - 1:
                rdma = pltpu.make_async_remote_copy(
                    src_ref=chunk_ref,
                    dst_ref=comm_ref.at[h],
                    send_sem=send_sems.at[h],
                    recv_sem=recv_sems.at[h],
                    device_id=(right,),
                    device_id_type=pl.DeviceIdType.MESH,
                )
                rdma.start()
            src_dev = jnp.mod(my - h, N_DEV)
            acc = contrib(chunk_ref[...], src_dev, acc)
            if h < N_DEV - 1:
                rdma.wait()

        out_ref[...] = acc

    return pl.pallas_call(
        body,
        out_shape=jax.ShapeDtypeStruct((n_tok, d_ff), jnp.float32),
        in_specs=[pl.BlockSpec(memory_space=pltpu.VMEM)] * 6,
        out_specs=pl.BlockSpec(memory_space=pltpu.VMEM),
        scratch_shapes=[
            pltpu.VMEM((N_DEV - 1, EXP_PER_DEV, d_model, d_ff), jnp.bfloat16),
            pltpu.SemaphoreType.DMA((N_DEV - 1,)),
            pltpu.SemaphoreType.DMA((N_DEV - 1,)),
            pltpu.SemaphoreType.DMA((N_DEV,)),
            pltpu.SemaphoreType.DMA((N_DEV,)),
            pltpu.VMEM((N_DEV, 128), jnp.float32),
        ],
        compiler_params=pltpu.CompilerParams(collective_id=0),
    )(xb, wb, route_f, oh, rank, cnts)


# baseline (device time: 175632 ns/iter reference)
import jax
import jax.numpy as jnp
from jax import lax
from jax.experimental import pallas as pl
from jax.experimental.pallas import tpu as pltpu

N_DEV = 4
N_EXP = 16
EXP_PER_DEV = 4
CAPACITY = 204


def kernel(x, router_W, route_idx, expert_W):
    del router_W
    n_tok, d_model = x.shape
    _, _, d_ff = expert_W.shape

    xb = x.astype(jnp.bfloat16)
    wb = expert_W.astype(jnp.bfloat16)
    route_f = route_idx.astype(jnp.float32)
    oh = (route_idx == jnp.arange(N_EXP, dtype=jnp.int32)[None, :]).astype(
        jnp.float32
    )
    rank = jnp.take_along_axis(
        jnp.cumsum(oh, axis=0) - oh, route_idx, axis=1
    )
    cnts = jnp.zeros((1, 128), jnp.float32).at[0, :N_EXP].set(oh.sum(axis=0))

    def body(
        x_ref, w_ref, route_ref, oh_ref, rank_ref, cnt_ref, out_ref,
        comm_ref, send_sems, recv_sems, cnt_send_sems, cnt_recv_sems,
        counts_all,
    ):
        my = lax.axis_index("i")
        right = jnp.mod(my + 1, N_DEV)

        barrier = pltpu.get_barrier_semaphore()
        for o in range(1, N_DEV):
            pl.semaphore_signal(
                barrier, inc=1,
                device_id=(jnp.mod(my + o, N_DEV),),
                device_id_type=pl.DeviceIdType.MESH,
            )
        pl.semaphore_wait(barrier, N_DEV - 1)

        cnt_sends = []
        for o in range(1, N_DEV):
            tgt = jnp.mod(my + o, N_DEV)
            s = pltpu.make_async_remote_copy(
                src_ref=cnt_ref,
                dst_ref=counts_all.at[pl.ds(my, 1)],
                send_sem=cnt_send_sems.at[o],
                recv_sem=cnt_recv_sems.at[my],
                device_id=(tgt,),
                device_id_type=pl.DeviceIdType.MESH,
            )
            s.start()
            cnt_sends.append(s)
        for o in range(1, N_DEV):
            src_dev = jnp.mod(my + o, N_DEV)
            r = pltpu.make_async_remote_copy(
                src_ref=cnt_ref,
                dst_ref=counts_all.at[pl.ds(src_dev, 1)],
                send_sem=cnt_send_sems.at[0],
                recv_sem=cnt_recv_sems.at[src_dev],
                device_id=(src_dev,),
                device_id_type=pl.DeviceIdType.MESH,
            )
            r.wait_recv()
        for s in cnt_sends:
            s.wait_send()

        ca = counts_all[...]
        dev_iota = lax.broadcasted_iota(jnp.int32, (N_DEV, 128), 0)
        prior = jnp.where(dev_iota < my, ca, 0.0).sum(
            axis=0, keepdims=True
        )
        offs_tok = (oh_ref[...] * prior[:, :N_EXP]).sum(
            axis=1, keepdims=True
        )
        keep = (offs_tok + rank_ref[...]) < float(CAPACITY)

        xv = x_ref[...]
        routev = route_ref[...]
        out_ref[...] = jnp.zeros((n_tok, d_ff), jnp.float32)

        for h in range(N_DEV):
            chunk_ref = w_ref if h == 0 else comm_ref.at[h - 1]
            if h < N_DEV - 1:
                rdma = pltpu.make_async_remote_copy(
                    src_ref=chunk_ref,
                    dst_ref=comm_ref.at[h],
                    send_sem=send_sems.at[h],
                    recv_sem=recv_sems.at[h],
                    device_id=(right,),
                    device_id_type=pl.DeviceIdType.MESH,
                )
                rdma.start()
            src_dev = jnp.mod(my - h, N_DEV)
            for k in range(EXP_PER_DEV):
                e = (src_dev * EXP_PER_DEV + k).astype(jnp.float32)
                m = jnp.logical_and(routev == e, keep)
                xm = xv * m.astype(jnp.bfloat16)
                out_ref[...] += jnp.dot(
                    xm, chunk_ref[k], preferred_element_type=jnp.float32
                )
            if h < N_DEV - 1:
                rdma.wait()

    return pl.pallas_call(
        body,
        out_shape=jax.ShapeDtypeStruct((n_tok, d_ff), jnp.float32),
        in_specs=[pl.BlockSpec(memory_space=pltpu.VMEM)] * 6,
        out_specs=pl.BlockSpec(memory_space=pltpu.VMEM),
        scratch_shapes=[
            pltpu.VMEM((N_DEV - 1, EXP_PER_DEV, d_model, d_ff), jnp.bfloat16),
            pltpu.SemaphoreType.DMA((N_DEV - 1,)),
            pltpu.SemaphoreType.DMA((N_DEV - 1,)),
            pltpu.SemaphoreType.DMA((N_DEV,)),
            pltpu.SemaphoreType.DMA((N_DEV,)),
            pltpu.VMEM((N_DEV, 128), jnp.float32),
        ],
        compiler_params=pltpu.CompilerParams(collective_id=0),
    )(xb, wb, route_f, oh, rank, cnts)


# device time: 105027 ns/iter; 1.6723x vs baseline; 1.6723x over previous
import jax
import jax.numpy as jnp
from jax import lax
from jax.experimental import pallas as pl
from jax.experimental.pallas import tpu as pltpu

N_DEV = 4
N_EXP = 16
EXP_PER_DEV = 4
CAPACITY = 204


def kernel(x, router_W, route_idx, expert_W):
    del router_W
    n_tok, d_model = x.shape
    _, _, d_ff = expert_W.shape

    xb = x.astype(jnp.bfloat16)
    wb = expert_W.astype(jnp.bfloat16)
    route_f = route_idx.astype(jnp.float32)
    oh = (route_idx == jnp.arange(N_EXP, dtype=jnp.int32)[None, :]).astype(
        jnp.float32
    )
    rank = jnp.take_along_axis(
        jnp.cumsum(oh, axis=0) - oh, route_idx, axis=1
    )
    cnts = jnp.zeros((1, 128), jnp.float32).at[0, :N_EXP].set(oh.sum(axis=0))

    def body(
        x_ref, w_ref, route_ref, oh_ref, rank_ref, cnt_ref, out_ref,
        comm_ref, send_sems, recv_sems, cnt_send_sems, cnt_recv_sems,
        counts_all,
    ):
        my = lax.axis_index("i")
        right = jnp.mod(my + 1, N_DEV)

        barrier = pltpu.get_barrier_semaphore()
        for o in range(1, N_DEV):
            pl.semaphore_signal(
                barrier, inc=1,
                device_id=(jnp.mod(my + o, N_DEV),),
                device_id_type=pl.DeviceIdType.MESH,
            )
        pl.semaphore_wait(barrier, N_DEV - 1)

        cnt_sends = []
        for o in range(1, N_DEV):
            tgt = jnp.mod(my + o, N_DEV)
            s = pltpu.make_async_remote_copy(
                src_ref=cnt_ref,
                dst_ref=counts_all.at[pl.ds(my, 1)],
                send_sem=cnt_send_sems.at[o],
                recv_sem=cnt_recv_sems.at[my],
                device_id=(tgt,),
                device_id_type=pl.DeviceIdType.MESH,
            )
            s.start()
            cnt_sends.append(s)

        xv = x_ref[...]
        routev = route_ref[...]
        left = jnp.mod(my - 1, N_DEV)

        def hop_acc(chunk_ref, src_dev):
            acc = None
            for k in range(EXP_PER_DEV):
                e = (src_dev * EXP_PER_DEV + k).astype(jnp.float32)
                xm = xv * (routev == e).astype(jnp.bfloat16)
                d = jnp.dot(xm, chunk_ref[k], preferred_element_type=jnp.float32)
                acc = d if acc is None else acc + d
            return acc

        r1f = pltpu.make_async_remote_copy(
            src_ref=w_ref, dst_ref=comm_ref.at[0],
            send_sem=send_sems.at[0], recv_sem=recv_sems.at[0],
            device_id=(right,), device_id_type=pl.DeviceIdType.MESH,
        )
        r1r = pltpu.make_async_remote_copy(
            src_ref=w_ref, dst_ref=comm_ref.at[1],
            send_sem=send_sems.at[1], recv_sem=recv_sems.at[1],
            device_id=(left,), device_id_type=pl.DeviceIdType.MESH,
        )
        r1f.start()
        r1r.start()

        out_ref[...] = hop_acc(w_ref, my)

        r1f.wait()
        r1r.wait()

        half = EXP_PER_DEV // 2
        r2f = pltpu.make_async_remote_copy(
            src_ref=comm_ref.at[0, pl.ds(0, half)],
            dst_ref=comm_ref.at[2, pl.ds(0, half)],
            send_sem=send_sems.at[2], recv_sem=recv_sems.at[2],
            device_id=(right,), device_id_type=pl.DeviceIdType.MESH,
        )
        r2r = pltpu.make_async_remote_copy(
            src_ref=comm_ref.at[1, pl.ds(half, half)],
            dst_ref=comm_ref.at[2, pl.ds(half, half)],
            send_sem=send_sems.at[3], recv_sem=recv_sems.at[3],
            device_id=(left,), device_id_type=pl.DeviceIdType.MESH,
        )
        r2f.start()
        r2r.start()

        out_ref[...] += hop_acc(comm_ref.at[0], jnp.mod(my - 1, N_DEV))
        out_ref[...] += hop_acc(comm_ref.at[1], jnp.mod(my + 1, N_DEV))

        r2f.wait()
        r2r.wait()

        out_ref[...] += hop_acc(comm_ref.at[2], jnp.mod(my + 2, N_DEV))

        for o in range(1, N_DEV):
            src_dev = jnp.mod(my + o, N_DEV)
            r = pltpu.make_async_remote_copy(
                src_ref=cnt_ref,
                dst_ref=counts_all.at[pl.ds(src_dev, 1)],
                send_sem=cnt_send_sems.at[0],
                recv_sem=cnt_recv_sems.at[src_dev],
                device_id=(src_dev,),
                device_id_type=pl.DeviceIdType.MESH,
            )
            r.wait_recv()
        for s in cnt_sends:
            s.wait_send()

        ca = counts_all[...]
        dev_iota = lax.broadcasted_iota(jnp.int32, (N_DEV, 128), 0)
        prior = jnp.where(dev_iota < my, ca, 0.0).sum(
            axis=0, keepdims=True
        )
        offs_tok = (oh_ref[...] * prior[:, :N_EXP]).sum(
            axis=1, keepdims=True
        )
        keep = (offs_tok + rank_ref[...]) < float(CAPACITY)
        out_ref[...] *= keep.astype(jnp.float32)

    return pl.pallas_call(
        body,
        out_shape=jax.ShapeDtypeStruct((n_tok, d_ff), jnp.float32),
        in_specs=[pl.BlockSpec(memory_space=pltpu.VMEM)] * 6,
        out_specs=pl.BlockSpec(memory_space=pltpu.VMEM),
        scratch_shapes=[
            pltpu.VMEM((N_DEV - 1, EXP_PER_DEV, d_model, d_ff), jnp.bfloat16),
            pltpu.SemaphoreType.DMA((4,)),
            pltpu.SemaphoreType.DMA((4,)),
            pltpu.SemaphoreType.DMA((N_DEV,)),
            pltpu.SemaphoreType.DMA((N_DEV,)),
            pltpu.VMEM((N_DEV, 128), jnp.float32),
        ],
        compiler_params=pltpu.CompilerParams(collective_id=0),
    )(xb, wb, route_f, oh, rank, cnts)
